# baseline (speedup 1.0000x reference)
import functools

import jax
import jax.numpy as jnp
from jax import lax
from jax.experimental import pallas as pl
from jax.experimental.pallas import tpu as pltpu
from jax.experimental.pallas import tpu_sc as plsc

MAX_POS = 8192
EMBED_DIM = 1024
BATCH = 4 * 4096

NUM_CORES = 2
NUM_SUBCORES = 16
NUM_WORKERS = NUM_CORES * NUM_SUBCORES
ROWS_PER_WORKER = BATCH // NUM_WORKERS
CHUNK = 16
NUM_CHUNKS = ROWS_PER_WORKER // CHUNK
NBUF = 4
NGROUPS = NUM_CHUNKS // NBUF


def _make_gather():
    mesh = plsc.VectorSubcoreMesh(core_axis_name="c", subcore_axis_name="s")

    @functools.partial(
        pl.kernel,
        mesh=mesh,
        out_type=jax.ShapeDtypeStruct((BATCH, EMBED_DIM), jnp.float32),
        scratch_types=[
            pltpu.VMEM((ROWS_PER_WORKER,), jnp.int32),
            pltpu.VMEM((NBUF, CHUNK, EMBED_DIM), jnp.float32),
            pltpu.SemaphoreType.DMA((NBUF,)),
            pltpu.SemaphoreType.DMA((NBUF,)),
        ],
    )
    def gather_kernel(x_hbm, table_hbm, out_hbm, idx_v, rows_v, gsem, ssem):
        wid = lax.axis_index("s") * NUM_CORES + lax.axis_index("c")
        base = wid * ROWS_PER_WORKER
        pltpu.sync_copy(x_hbm.at[pl.ds(base, ROWS_PER_WORKER)], idx_v)

        def gather(c, b):
            off = pl.multiple_of(c * CHUNK, 8)
            return pltpu.async_copy(
                table_hbm.at[idx_v.at[pl.ds(off, CHUNK)]],
                rows_v.at[b],
                gsem.at[b],
            )

        def store(c, b):
            off = pl.multiple_of(base + c * CHUNK, 8)
            return pltpu.async_copy(
                rows_v.at[b],
                out_hbm.at[pl.ds(off, CHUNK)],
                ssem.at[b],
            )

        def wait_gather(b):
            pltpu.make_async_copy(
                table_hbm.at[idx_v.at[pl.ds(0, CHUNK)]],
                rows_v.at[b],
                gsem.at[b],
            ).wait()

        def wait_store(b):
            pltpu.make_async_copy(
                rows_v.at[b],
                out_hbm.at[pl.ds(0, CHUNK)],
                ssem.at[b],
            ).wait()

        for b in range(NBUF):
            gather(b, b)

        def body(g, _):
            for b in range(NBUF):
                wait_gather(b)
                store(g * NBUF + b, b)
            @pl.when(g < NGROUPS - 1)
            def _refill():
                for b in range(NBUF):
                    wait_store(b)
                    gather((g + 1) * NBUF + b, b)
            return None

        lax.fori_loop(0, NGROUPS, body, None)
        for b in range(NBUF):
            wait_store(b)

    return gather_kernel


_gather = _make_gather()


@jax.jit
def kernel(x, embeddings):
    flat = x.reshape(BATCH)
    out = _gather(flat, embeddings)
    return out.reshape(x.shape + (EMBED_DIM,))


# speedup vs baseline: 1.0382x; 1.0382x over previous
import functools

import jax
import jax.numpy as jnp
from jax import lax
from jax.experimental import pallas as pl
from jax.experimental.pallas import tpu as pltpu
from jax.experimental.pallas import tpu_sc as plsc

MAX_POS = 8192
EMBED_DIM = 1024
ROWS = 4
COLS = 4096
BATCH = ROWS * COLS

NUM_CORES = 2
NUM_SUBCORES = 16
NUM_WORKERS = NUM_CORES * NUM_SUBCORES
ROWS_PER_WORKER = BATCH // NUM_WORKERS
WORKERS_PER_XROW = COLS // ROWS_PER_WORKER
CHUNK = 16
NUM_CHUNKS = ROWS_PER_WORKER // CHUNK
NBUF = 6


def _make_gather():
    mesh = plsc.VectorSubcoreMesh(core_axis_name="c", subcore_axis_name="s")

    @functools.partial(
        pl.kernel,
        mesh=mesh,
        out_type=jax.ShapeDtypeStruct((ROWS, COLS, EMBED_DIM), jnp.float32),
        scratch_types=[
            pltpu.VMEM((ROWS_PER_WORKER,), jnp.int32),
            pltpu.VMEM((NBUF, CHUNK, EMBED_DIM), jnp.float32),
            pltpu.SemaphoreType.DMA((NBUF,)),
            pltpu.SemaphoreType.DMA((NBUF,)),
        ],
    )
    def gather_kernel(x_hbm, table_hbm, out_hbm, idx_v, rows_v, gsem, ssem):
        wid = lax.axis_index("s") * NUM_CORES + lax.axis_index("c")
        xrow = wid // WORKERS_PER_XROW
        col0 = (wid % WORKERS_PER_XROW) * ROWS_PER_WORKER
        pltpu.sync_copy(x_hbm.at[xrow, pl.ds(col0, ROWS_PER_WORKER)], idx_v)

        def gather(k):
            b = k % NBUF
            return pltpu.async_copy(
                table_hbm.at[idx_v.at[pl.ds(k * CHUNK, CHUNK)]],
                rows_v.at[b],
                gsem.at[b],
            )

        def store(k):
            b = k % NBUF
            return pltpu.async_copy(
                rows_v.at[b],
                out_hbm.at[xrow, pl.ds(col0 + k * CHUNK, CHUNK)],
                ssem.at[b],
            )

        A = NBUF - 2
        g_descs = [None] * NUM_CHUNKS
        s_descs = [None] * NUM_CHUNKS
        for j in range(A):
            g_descs[j] = gather(j)
        for k in range(NUM_CHUNKS):
            j = k + A
            if j < NUM_CHUNKS:
                if j - NBUF >= 0:
                    s_descs[j - NBUF].wait()
                g_descs[j] = gather(j)
            g_descs[k].wait()
            s_descs[k] = store(k)
        for k in range(max(0, NUM_CHUNKS - NBUF), NUM_CHUNKS):
            s_descs[k].wait()

    return gather_kernel


_gather = _make_gather()


@jax.jit
def kernel(x, embeddings):
    return _gather(x, embeddings)
